# baseline (speedup 1.0000x reference)
import jax
import jax.numpy as jnp
from jax import lax
from jax.experimental import pallas as pl
from jax.experimental.pallas import tpu as pltpu
from jax.experimental.pallas import tpu_sc as plsc

N_CLASSES = 1000
LB_SMOOTH = 0.1
LB_POS = 1.0 - LB_SMOOTH
LB_NEG = LB_SMOOTH / N_CLASSES

B = 16384
NW = 32
COLS_PER_W = B // NW
CC = 512
COL_CHUNKS = COLS_PER_W // CC
QROWS = 64
QLENS = (64,) * 15 + (40,)
QOFFS = tuple(64 * i for i in range(16))


def _onehot_body(label_hbm, out_hbm, lbl_v, buf0, buf1, sem0, sem1):
    wid = lax.axis_index("s") * 2 + lax.axis_index("c")
    base_col = wid * COLS_PER_W

    pltpu.sync_copy(label_hbm.at[pl.ds(base_col, COLS_PER_W)], lbl_v)

    neg = jnp.full((16,), LB_NEG, dtype=jnp.float32)
    pos = jnp.full((16,), LB_POS, dtype=jnp.float32)
    lane = lax.iota(jnp.int32, 16)

    def fill_both(k, _):
        row = k >> 5
        off = (k & 31) * 16
        buf0[row, pl.ds(off, 16)] = neg
        buf1[row, pl.ds(off, 16)] = neg
        return 0
    lax.fori_loop(0, QROWS * CC // 16, fill_both, 0)

    def scatter_hot(buf, q, cc, val):
        for j in range(CC // 16):
            lbl = lbl_v[pl.ds(cc * CC + j * 16, 16)]
            row = lbl - QOFFS[q]
            mask = (row >= 0) & (row < QLENS[q])
            plsc.store_scatter(buf, [row, lane + j * 16], val, mask=mask)

    bufs = (buf0, buf1)
    sems = (sem0, sem1)
    copies = [None, None]
    prev = [None, None]

    def run_chunk(n, q, cc):
        b = n & 1
        buf = bufs[b]
        if copies[b] is not None:
            copies[b].wait()
            scatter_hot(buf, *prev[b], neg)
        scatter_hot(buf, q, cc, pos)
        prev[b] = (q, cc)
        copies[b] = pltpu.async_copy(
            buf.at[pl.ds(0, QLENS[q]), :],
            out_hbm.at[pl.ds(QOFFS[q], QLENS[q]),
                       pl.ds(base_col + cc * CC, CC)],
            sems[b],
        )

    chunks = [(q, cc) for cc in range(COL_CHUNKS) for q in range(len(QLENS))]
    for n, (q, cc) in enumerate(chunks):
        run_chunk(n, q, cc)
    copies[0].wait()
    copies[1].wait()


@jax.jit
def kernel(label):
    mesh = plsc.VectorSubcoreMesh(core_axis_name="c", subcore_axis_name="s")
    out_t = pl.kernel(
        _onehot_body,
        out_type=jax.ShapeDtypeStruct((N_CLASSES, B), jnp.float32),
        mesh=mesh,
        scratch_types=[
            pltpu.VMEM((COLS_PER_W,), jnp.int32),
            pltpu.VMEM((QROWS, CC), jnp.float32),
            pltpu.VMEM((QROWS, CC), jnp.float32),
            pltpu.SemaphoreType.DMA,
            pltpu.SemaphoreType.DMA,
        ],
        compiler_params=pltpu.CompilerParams(
            needs_layout_passes=False, use_tc_tiling_on_sc=True),
    )(label)
    return out_t.T


# speedup vs baseline: 1.3016x; 1.3016x over previous
import jax
import jax.numpy as jnp
from jax import lax
from jax.experimental import pallas as pl
from jax.experimental.pallas import tpu as pltpu
from jax.experimental.pallas import tpu_sc as plsc

N_CLASSES = 1000
LB_SMOOTH = 0.1
LB_POS = 1.0 - LB_SMOOTH
LB_NEG = LB_SMOOTH / N_CLASSES

B = 16384
NW = 32
COLS_PER_W = B // NW
CC = 128
COL_CHUNKS = COLS_PER_W // CC
QROWS = 256
QLENS = (256, 256, 256, 232)
QOFFS = (0, 256, 512, 768)


def _onehot_body(label_hbm, out_hbm, lbl_v, buf0, buf1, sem0, sem1):
    wid = lax.axis_index("s") * 2 + lax.axis_index("c")
    base_col = wid * COLS_PER_W

    pltpu.sync_copy(label_hbm.at[pl.ds(base_col, COLS_PER_W)], lbl_v)

    neg = jnp.full((16,), LB_NEG, dtype=jnp.float32)
    pos = jnp.full((16,), LB_POS, dtype=jnp.float32)
    lane = lax.iota(jnp.int32, 16)

    @plsc.parallel_loop(0, QROWS * CC // 16, unroll=8)
    def fill_both(k):
        row = k >> 3
        off = (k & 7) * 16
        buf0[row, pl.ds(off, 16)] = neg
        buf1[row, pl.ds(off, 16)] = neg

    def scatter_hot(buf, q, cc, val):
        for j in range(CC // 16):
            lbl = lbl_v[pl.ds(cc * CC + j * 16, 16)]
            row = lbl - QOFFS[q]
            mask = (row >= 0) & (row < QLENS[q])
            plsc.store_scatter(buf, [row, lane + j * 16], val, mask=mask)

    bufs = (buf0, buf1)
    sems = (sem0, sem1)
    copies = [None, None]
    prev = [None, None]

    def run_chunk(n, q, cc):
        b = n & 1
        buf = bufs[b]
        if copies[b] is not None:
            copies[b].wait()
            scatter_hot(buf, *prev[b], neg)
        scatter_hot(buf, q, cc, pos)
        prev[b] = (q, cc)
        copies[b] = pltpu.async_copy(
            buf.at[pl.ds(0, QLENS[q]), :],
            out_hbm.at[pl.ds(QOFFS[q], QLENS[q]),
                       pl.ds(base_col + cc * CC, CC)],
            sems[b],
        )

    chunks = [(q, cc) for cc in range(COL_CHUNKS) for q in range(len(QLENS))]
    for n, (q, cc) in enumerate(chunks):
        run_chunk(n, q, cc)
    copies[0].wait()
    copies[1].wait()


@jax.jit
def kernel(label):
    mesh = plsc.VectorSubcoreMesh(core_axis_name="c", subcore_axis_name="s")
    out_t = pl.kernel(
        _onehot_body,
        out_type=jax.ShapeDtypeStruct((N_CLASSES, B), jnp.float32),
        mesh=mesh,
        scratch_types=[
            pltpu.VMEM((COLS_PER_W,), jnp.int32),
            pltpu.VMEM((QROWS, CC), jnp.float32),
            pltpu.VMEM((QROWS, CC), jnp.float32),
            pltpu.SemaphoreType.DMA,
            pltpu.SemaphoreType.DMA,
        ],
        compiler_params=pltpu.CompilerParams(
            needs_layout_passes=False, use_tc_tiling_on_sc=True),
    )(label)
    return out_t.T


# speedup vs baseline: 1.3296x; 1.0216x over previous
import jax
import jax.numpy as jnp
from jax import lax
from jax.experimental import pallas as pl
from jax.experimental.pallas import tpu as pltpu
from jax.experimental.pallas import tpu_sc as plsc

N_CLASSES = 1000
LB_SMOOTH = 0.1
LB_POS = 1.0 - LB_SMOOTH
LB_NEG = LB_SMOOTH / N_CLASSES

B = 16384
NW = 32
COLS_PER_W = B // NW
CC = 128
COL_CHUNKS = COLS_PER_W // CC
QROWS = 256
QLENS = (256, 256, 256, 232)
QOFFS = (0, 256, 512, 768)


def _onehot_body(label_hbm, out_hbm, lbl_v, buf0, buf1, sem0, sem1):
    wid = lax.axis_index("s") * 2 + lax.axis_index("c")
    base_col = wid * COLS_PER_W

    pltpu.sync_copy(label_hbm.at[pl.ds(base_col, COLS_PER_W)], lbl_v)

    neg = jnp.full((16,), LB_NEG, dtype=jnp.float32)
    pos = jnp.full((16,), LB_POS, dtype=jnp.float32)
    lane = lax.iota(jnp.int32, 16)

    def fill(buf):
        @plsc.parallel_loop(0, QROWS * CC // 16, unroll=8)
        def body(k):
            buf[k >> 3, pl.ds((k & 7) * 16, 16)] = neg

    def scatter_hot(buf, q, cc, val):
        for j in range(CC // 16):
            lbl = lbl_v[pl.ds(cc * CC + j * 16, 16)]
            row = lbl - QOFFS[q]
            mask = (row >= 0) & (row < QLENS[q])
            plsc.store_scatter(buf, [row, lane + j * 16], val, mask=mask)

    bufs = (buf0, buf1)
    sems = (sem0, sem1)
    copies = [None, None]
    prev = [None, None]

    def run_chunk(n, q, cc):
        b = n & 1
        buf = bufs[b]
        if copies[b] is not None:
            copies[b].wait()
            scatter_hot(buf, *prev[b], neg)
        scatter_hot(buf, q, cc, pos)
        prev[b] = (q, cc)
        copies[b] = pltpu.async_copy(
            buf.at[pl.ds(0, QLENS[q]), :],
            out_hbm.at[pl.ds(QOFFS[q], QLENS[q]),
                       pl.ds(base_col + cc * CC, CC)],
            sems[b],
        )

    chunks = [(q, cc) for cc in range(COL_CHUNKS) for q in range(len(QLENS))]
    fill(buf0)
    run_chunk(0, *chunks[0])
    fill(buf1)
    for n, (q, cc) in enumerate(chunks[1:], start=1):
        run_chunk(n, q, cc)
    copies[0].wait()
    copies[1].wait()


@jax.jit
def kernel(label):
    mesh = plsc.VectorSubcoreMesh(core_axis_name="c", subcore_axis_name="s")
    out_t = pl.kernel(
        _onehot_body,
        out_type=jax.ShapeDtypeStruct((N_CLASSES, B), jnp.float32),
        mesh=mesh,
        scratch_types=[
            pltpu.VMEM((COLS_PER_W,), jnp.int32),
            pltpu.VMEM((QROWS, CC), jnp.float32),
            pltpu.VMEM((QROWS, CC), jnp.float32),
            pltpu.SemaphoreType.DMA,
            pltpu.SemaphoreType.DMA,
        ],
        compiler_params=pltpu.CompilerParams(
            needs_layout_passes=False, use_tc_tiling_on_sc=True),
    )(label)
    return out_t.T


# speedup vs baseline: 1.3381x; 1.0064x over previous
import jax
import jax.numpy as jnp
from jax import lax
from jax.experimental import pallas as pl
from jax.experimental.pallas import tpu as pltpu
from jax.experimental.pallas import tpu_sc as plsc

N_CLASSES = 1000
LB_SMOOTH = 0.1
LB_POS = 1.0 - LB_SMOOTH
LB_NEG = LB_SMOOTH / N_CLASSES

B = 16384
NW = 32
COLS_PER_W = B // NW
CC = 128
COL_CHUNKS = COLS_PER_W // CC
QROWS = 256
QLENS = (256, 256, 256, 232)
QOFFS = (0, 256, 512, 768)


def _onehot_body(label_hbm, out_hbm, lbl_v, buf0, buf1, sem0, sem1):
    wid = lax.axis_index("s") * 2 + lax.axis_index("c")
    base_col = wid * COLS_PER_W

    pltpu.sync_copy(label_hbm.at[pl.ds(base_col, COLS_PER_W)], lbl_v)

    neg = jnp.full((16,), LB_NEG, dtype=jnp.float32)
    pos = jnp.full((16,), LB_POS, dtype=jnp.float32)
    lane = lax.iota(jnp.int32, 16)

    def fill(buf):
        @plsc.parallel_loop(0, QROWS * CC // 16, unroll=8)
        def body(k):
            buf[k >> 3, pl.ds((k & 7) * 16, 16)] = neg

    def scatter_hot(buf, q, cc, val):
        for j in range(CC // 16):
            lbl = lbl_v[pl.ds(cc * CC + j * 16, 16)]
            row = lbl - QOFFS[q]
            mask = (row >= 0) & (row < QLENS[q])
            plsc.store_scatter(buf, [row, lane + j * 16], val, mask=mask)

    bufs = (buf0, buf1)
    sems = (sem0, sem1)
    copies = [None, None]
    prev = [None, None]

    def run_chunk(n, q, cc):
        b = n & 1
        buf = bufs[b]
        if copies[b] is not None:
            copies[b].wait()
            scatter_hot(buf, *prev[b], neg)
        scatter_hot(buf, q, cc, pos)
        prev[b] = (q, cc)
        copies[b] = pltpu.async_copy(
            buf.at[pl.ds(0, QLENS[q]), :],
            out_hbm.at[pl.ds(QOFFS[q], QLENS[q]),
                       pl.ds(base_col + cc * CC, CC)],
            sems[b],
        )

    chunks = [(q, cc) for cc in range(COL_CHUNKS) for q in range(len(QLENS))]
    fill(buf0)
    run_chunk(0, *chunks[0])
    fill(buf1)
    for n, (q, cc) in enumerate(chunks[1:], start=1):
        run_chunk(n, q, cc)
    copies[0].wait()
    copies[1].wait()


@jax.jit
def kernel(label):
    mesh = plsc.VectorSubcoreMesh(core_axis_name="c", subcore_axis_name="s")
    out_t = pl.kernel(
        _onehot_body,
        out_type=jax.ShapeDtypeStruct((N_CLASSES, B), jnp.float32),
        mesh=mesh,
        scratch_types=[
            pltpu.VMEM((COLS_PER_W,), jnp.int32),
            pltpu.VMEM((QROWS, CC), jnp.float32),
            pltpu.VMEM((QROWS, CC), jnp.float32),
            pltpu.SemaphoreType.DMA,
            pltpu.SemaphoreType.DMA,
        ],
        compiler_params=pltpu.CompilerParams(
            needs_layout_passes=False, use_tc_tiling_on_sc=True,
            disable_bounds_checks=True, disable_semaphore_checks=True),
    )(label)
    return out_t.T
